# Initial kernel scaffold; baseline (speedup 1.0000x reference)
import functools

import jax
import jax.numpy as jnp
from jax import lax
from jax.experimental import pallas as pl
from jax.experimental.pallas import tpu as pltpu
from jax.experimental.pallas import tpu_sc as plsc

NN = 10000
EE = 320000
D_IN = 128
D_HID = 64
D_OUT = 16
NC = 2
NS = 16
ET = EE // (NC * NS)
ES = EE // NS
CH = 80
RPT = NN // NS
RB = 125

_MESH = plsc.VectorSubcoreMesh(core_axis_name="c", subcore_axis_name="s")

_F32 = jnp.float32
_ZV = jnp.zeros((16,), _F32)


def _fill_zero_2d(ref, rows, width):
    def body(i, _):
        for k in range(width // 16):
            ref[i, pl.ds(k * 16, 16)] = _ZV
        return 0
    lax.fori_loop(0, rows, body, 0)


def _fill_zero_1d(ref, n):
    def body(i, _):
        ref[pl.ds(i * 16, 16)] = _ZV
        return 0
    lax.fori_loop(0, n // 16, body, 0)


def _newton_rsqrt(x):
    i = lax.bitcast_convert_type(x, jnp.int32)
    y = lax.bitcast_convert_type(jnp.int32(0x5F3759DF) - (i >> 1), _F32)
    for _ in range(3):
        y = y * (1.5 - 0.5 * x * y * y)
    return jnp.where(x > 0, y, 0.0)


def _stage_rows_to_spmem(hbm, sh, bounce, s, width):
    def body(i, _):
        r = s * RPT + i * RB
        pltpu.sync_copy(hbm.at[pl.ds(r, RB), :], bounce)
        pltpu.sync_copy(bounce, sh.at[pl.ds(r, RB), :])
        return 0
    lax.fori_loop(0, RPT // RB, body, 0)


def _export_spmem_rows(sh, hbm_out, bounce, c, s):
    def body(i, _):
        r = s * RPT + i * RB
        pltpu.sync_copy(sh.at[pl.ds(r, RB), :], bounce)
        pltpu.sync_copy(bounce, hbm_out.at[c, pl.ds(r, RB), :])
        return 0
    lax.fori_loop(0, RPT // RB, body, 0)


def _zero_acc_rows(acc_sh, zrows, s):
    def body(i, _):
        pltpu.sync_copy(zrows, acc_sh.at[pl.ds(s * RPT + i * RB, RB), :])
        return 0
    lax.fori_loop(0, RPT // RB, body, 0)


def _prop_chunks(src_v, dst_v, eoff, norm_v, y_sh, acc_sh, sidx, didx, rows,
                 width):
    nw = width // 16

    def chunk(j, _):
        e0 = j * CH
        for k in range(CH // 16):
            sidx[pl.ds(k * 16, 16)] = src_v[pl.ds(eoff + e0 + k * 16, 16)]
            didx[pl.ds(k * 16, 16)] = dst_v[pl.ds(eoff + e0 + k * 16, 16)]
        pltpu.sync_copy(y_sh.at[sidx], rows)

        def edge(e, _):
            ns = norm_v[e0 + e]
            for k in range(nw):
                rows[e, pl.ds(k * 16, 16)] = rows[e, pl.ds(k * 16, 16)] * ns
            return 0
        lax.fori_loop(0, CH, edge, 0)
        pltpu.sync_copy(rows, acc_sh.at[didx], add=True)
        return 0
    lax.fori_loop(0, ET // CH, chunk, 0)


def _sc1_body(src_h, dst_h, ew_h, c1_h, norm_h, u_h,
              eb_src, eb_dst, eb_ew, vbuf, sidx, didx, dsv, ddv, norm_v,
              rows, bounce, zrows, zflat,
              deg_sh, dis_sh, c1_sh, u_sh):
    c = lax.axis_index("c")
    s = lax.axis_index("s")

    pltpu.sync_copy(src_h.at[pl.ds(s * ES, ES)], eb_src)
    pltpu.sync_copy(dst_h.at[pl.ds(s * ES, ES)], eb_dst)
    pltpu.sync_copy(ew_h.at[pl.ds(s * ES, ES)], eb_ew)
    _fill_zero_2d(zrows, RB, D_HID)
    _zero_acc_rows(u_sh, zrows, s)
    _stage_rows_to_spmem(c1_h, c1_sh, bounce, s, D_HID)

    @pl.when(s < 5)
    def _():
        _fill_zero_1d(zflat, 2000)
        pltpu.sync_copy(zflat, deg_sh.at[pl.ds(s * 2000, 2000)])

    plsc.subcore_barrier()

    def deg_chunk(j, _):
        e0 = j * CH
        for k in range(CH // 16):
            sv = eb_src[pl.ds(e0 + k * 16, 16)]
            dv = eb_dst[pl.ds(e0 + k * 16, 16)]
            wv = eb_ew[pl.ds(e0 + k * 16, 16)]
            vbuf[pl.ds(k * 16, 16)] = jnp.where(sv == dv, 0.0, wv)
            sidx[pl.ds(k * 16, 16)] = sv
        pltpu.sync_copy(vbuf, deg_sh.at[sidx], add=True)
        return 0
    lax.fori_loop(0, ES // CH, deg_chunk, 0)

    plsc.subcore_barrier()

    def dis_slice(r0, n):
        pltpu.sync_copy(deg_sh.at[pl.ds(r0, n)], norm_v.at[pl.ds(0, n)])

        def body(i, _):
            v = norm_v[pl.ds(i * 16, 16)]
            norm_v[pl.ds(i * 16, 16)] = _newton_rsqrt(v)
            return 0
        lax.fori_loop(0, n // 16, body, 0)
        pltpu.sync_copy(norm_v.at[pl.ds(0, n)], dis_sh.at[pl.ds(r0, n)])

    @pl.when(s < 15)
    def _():
        dis_slice(s * 640, 640)

    @pl.when(s == 15)
    def _():
        dis_slice(9600, 400)

    plsc.subcore_barrier()

    eoff = c * ET

    def chunk(j, _):
        e0 = eoff + j * CH
        for k in range(CH // 16):
            sidx[pl.ds(k * 16, 16)] = eb_src[pl.ds(e0 + k * 16, 16)]
            didx[pl.ds(k * 16, 16)] = eb_dst[pl.ds(e0 + k * 16, 16)]
        pltpu.sync_copy(dis_sh.at[sidx], dsv)
        pltpu.sync_copy(dis_sh.at[didx], ddv)
        for k in range(CH // 16):
            sv = sidx[pl.ds(k * 16, 16)]
            dv = didx[pl.ds(k * 16, 16)]
            wv = eb_ew[pl.ds(e0 + k * 16, 16)]
            wv = jnp.where(sv == dv, 0.0, wv)
            nv = -(dsv[pl.ds(k * 16, 16)] * wv * ddv[pl.ds(k * 16, 16)])
            norm_v[pl.ds(j * CH + k * 16, 16)] = nv
        pltpu.sync_copy(c1_sh.at[sidx], rows)

        def edge(e, _):
            ns = norm_v[j * CH + e]
            for k in range(D_HID // 16):
                rows[e, pl.ds(k * 16, 16)] = rows[e, pl.ds(k * 16, 16)] * ns
            return 0
        lax.fori_loop(0, CH, edge, 0)
        pltpu.sync_copy(rows, u_sh.at[didx], add=True)
        return 0
    lax.fori_loop(0, ET // CH, chunk, 0)

    wid = s * NC + c
    pltpu.sync_copy(norm_v, norm_h.at[pl.ds(wid * ET, ET)])

    plsc.subcore_barrier()
    _export_spmem_rows(u_sh, u_h, bounce, c, s)


def _sc1(src, dst, ew, c1):
    f = pl.kernel(
        _sc1_body,
        out_type=[
            jax.ShapeDtypeStruct((EE,), _F32),
            jax.ShapeDtypeStruct((NC, NN, D_HID), _F32),
        ],
        mesh=_MESH,
        scratch_types=[
            pltpu.VMEM((ES,), jnp.int32),
            pltpu.VMEM((ES,), jnp.int32),
            pltpu.VMEM((ES,), _F32),
            pltpu.VMEM((CH,), _F32),
            pltpu.VMEM((CH,), jnp.int32),
            pltpu.VMEM((CH,), jnp.int32),
            pltpu.VMEM((CH,), _F32),
            pltpu.VMEM((CH,), _F32),
            pltpu.VMEM((ET,), _F32),
            pltpu.VMEM((CH, D_HID), _F32),
            pltpu.VMEM((RB, D_HID), _F32),
            pltpu.VMEM((RB, D_HID), _F32),
            pltpu.VMEM((2000,), _F32),
            pltpu.VMEM_SHARED((NN,), _F32),
            pltpu.VMEM_SHARED((NN,), _F32),
            pltpu.VMEM_SHARED((NN, D_HID), _F32),
            pltpu.VMEM_SHARED((NN, D_HID), _F32),
        ],
    )
    return f(src, dst, ew, c1)


def _make_prop_second(width):
    def body(src_h, dst_h, bx_h, u_h, norm_h, v_out,
             es_v, ed_v, norm_v, sidx, didx, rows, bounce, ub0, ub1, zrows,
             z_sh, v_sh):
        c = lax.axis_index("c")
        s = lax.axis_index("s")
        wid = s * NC + c

        pltpu.sync_copy(src_h.at[pl.ds(wid * ET, ET)], es_v)
        pltpu.sync_copy(dst_h.at[pl.ds(wid * ET, ET)], ed_v)
        pltpu.sync_copy(norm_h.at[pl.ds(wid * ET, ET)], norm_v)
        _fill_zero_2d(zrows, RB, width)
        _zero_acc_rows(v_sh, zrows, s)

        def zslice(i, _):
            r = s * RPT + i * RB
            pltpu.sync_copy(bx_h.at[pl.ds(r, RB), :], bounce)
            pltpu.sync_copy(u_h.at[0, pl.ds(r, RB), :], ub0)
            pltpu.sync_copy(u_h.at[1, pl.ds(r, RB), :], ub1)

            def row(i2, _):
                for k in range(width // 16):
                    d = pl.ds(k * 16, 16)
                    bounce[i2, d] = (bounce[i2, d]
                                     + 2.0 * (ub0[i2, d] + ub1[i2, d]))
                return 0
            lax.fori_loop(0, RB, row, 0)
            pltpu.sync_copy(bounce, z_sh.at[pl.ds(r, RB), :])
            return 0
        lax.fori_loop(0, RPT // RB, zslice, 0)

        plsc.subcore_barrier()
        _prop_chunks(es_v, ed_v, 0, norm_v, z_sh, v_sh, sidx, didx, rows,
                     width)
        plsc.subcore_barrier()
        _export_spmem_rows(v_sh, v_out, bounce, c, s)

    def run(src, dst, bx, u, norm):
        f = pl.kernel(
            body,
            out_type=[jax.ShapeDtypeStruct((NC, NN, width), _F32)],
            mesh=_MESH,
            scratch_types=[
                pltpu.VMEM((ET,), jnp.int32),
                pltpu.VMEM((ET,), jnp.int32),
                pltpu.VMEM((ET,), _F32),
                pltpu.VMEM((CH,), jnp.int32),
                pltpu.VMEM((CH,), jnp.int32),
                pltpu.VMEM((CH, width), _F32),
                pltpu.VMEM((RB, width), _F32),
                pltpu.VMEM((RB, width), _F32),
                pltpu.VMEM((RB, width), _F32),
                pltpu.VMEM((RB, width), _F32),
                pltpu.VMEM_SHARED((NN, width), _F32),
                pltpu.VMEM_SHARED((NN, width), _F32),
            ],
        )
        return f(src, dst, bx, u, norm)[0]
    return run


def _make_prop_first(width):
    def body(src_h, dst_h, y_h, norm_h, u_out,
             es_v, ed_v, norm_v, sidx, didx, rows, bounce, zrows,
             y_sh, acc_sh):
        c = lax.axis_index("c")
        s = lax.axis_index("s")
        wid = s * NC + c

        pltpu.sync_copy(src_h.at[pl.ds(wid * ET, ET)], es_v)
        pltpu.sync_copy(dst_h.at[pl.ds(wid * ET, ET)], ed_v)
        pltpu.sync_copy(norm_h.at[pl.ds(wid * ET, ET)], norm_v)
        _fill_zero_2d(zrows, RB, width)
        _zero_acc_rows(acc_sh, zrows, s)
        _stage_rows_to_spmem(y_h, y_sh, bounce, s, width)

        plsc.subcore_barrier()
        _prop_chunks(es_v, ed_v, 0, norm_v, y_sh, acc_sh, sidx, didx, rows,
                     width)
        plsc.subcore_barrier()
        _export_spmem_rows(acc_sh, u_out, bounce, c, s)

    def run(src, dst, y, norm):
        f = pl.kernel(
            body,
            out_type=[jax.ShapeDtypeStruct((NC, NN, width), _F32)],
            mesh=_MESH,
            scratch_types=[
                pltpu.VMEM((ET,), jnp.int32),
                pltpu.VMEM((ET,), jnp.int32),
                pltpu.VMEM((ET,), _F32),
                pltpu.VMEM((CH,), jnp.int32),
                pltpu.VMEM((CH,), jnp.int32),
                pltpu.VMEM((CH, width), _F32),
                pltpu.VMEM((RB, width), _F32),
                pltpu.VMEM((RB, width), _F32),
                pltpu.VMEM_SHARED((NN, width), _F32),
                pltpu.VMEM_SHARED((NN, width), _F32),
            ],
        )
        return f(src, dst, y, norm)[0]
    return run


_prop_second_64 = _make_prop_second(D_HID)
_prop_first_16 = _make_prop_first(D_OUT)
_prop_second_16 = _make_prop_second(D_OUT)


_BN = 1250


def _tc_matmul_split(x, w, d_in, d_out):
    def body(x_ref, w_ref, a_ref, b_ref, c_ref):
        y = jnp.dot(x_ref[...], w_ref[...], preferred_element_type=_F32)
        a_ref[...] = y[:, :d_out]
        b_ref[...] = y[:, d_out:2 * d_out]
        c_ref[...] = y[:, 2 * d_out:]

    out = pl.pallas_call(
        body,
        grid=(NN // _BN,),
        in_specs=[
            pl.BlockSpec((_BN, d_in), lambda i: (i, 0)),
            pl.BlockSpec((d_in, 3 * d_out), lambda i: (0, 0)),
        ],
        out_specs=[pl.BlockSpec((_BN, d_out), lambda i: (i, 0))] * 3,
        out_shape=[jax.ShapeDtypeStruct((NN, d_out), _F32)] * 3,
    )(x, w)
    return out


def _tc_layer2_head(a1, v, b1, w2):
    def body(a_ref, v_ref, b_ref, w_ref, a2_ref, b2_ref, c2_ref):
        h = a_ref[...] + v_ref[0] + v_ref[1] + b_ref[...]
        h = jnp.maximum(h, 0.0)
        z = jnp.dot(h, w_ref[...], preferred_element_type=_F32)
        a2_ref[...] = z[:, :D_OUT]
        b2_ref[...] = z[:, D_OUT:2 * D_OUT]
        c2_ref[...] = z[:, 2 * D_OUT:]

    out = pl.pallas_call(
        body,
        grid=(NN // _BN,),
        in_specs=[
            pl.BlockSpec((_BN, D_HID), lambda i: (i, 0)),
            pl.BlockSpec((NC, _BN, D_HID), lambda i: (0, i, 0)),
            pl.BlockSpec((1, D_HID), lambda i: (0, 0)),
            pl.BlockSpec((D_HID, 3 * D_OUT), lambda i: (0, 0)),
        ],
        out_specs=[pl.BlockSpec((_BN, D_OUT), lambda i: (i, 0))] * 3,
        out_shape=[jax.ShapeDtypeStruct((NN, D_OUT), _F32)] * 3,
    )(a1, v, b1, w2)
    return out


def _tc_final(a2, v2, b2):
    def body(a_ref, v_ref, b_ref, o_ref):
        o_ref[...] = a_ref[...] + v_ref[0] + v_ref[1] + b_ref[...]

    return pl.pallas_call(
        body,
        grid=(NN // _BN,),
        in_specs=[
            pl.BlockSpec((_BN, D_OUT), lambda i: (i, 0)),
            pl.BlockSpec((NC, _BN, D_OUT), lambda i: (0, i, 0)),
            pl.BlockSpec((1, D_OUT), lambda i: (0, 0)),
        ],
        out_specs=pl.BlockSpec((_BN, D_OUT), lambda i: (i, 0)),
        out_shape=jax.ShapeDtypeStruct((NN, D_OUT), _F32),
    )(a2, v2, b2)


def kernel(x, edge_index, edge_weight, W1, b1, W2, b2):
    src = edge_index[0]
    dst = edge_index[1]

    w1a, w1b, w1c = W1[:D_IN], W1[D_IN:2 * D_IN], W1[2 * D_IN:]
    wcat1 = jnp.concatenate([w1a - w1c, w1b, w1c], axis=1)
    w2a, w2b, w2c = W2[:D_HID], W2[D_HID:2 * D_HID], W2[2 * D_HID:]
    wcat2 = jnp.concatenate([w2a - w2c, w2b, w2c], axis=1)

    a1, b1x, c1 = _tc_matmul_split(x, wcat1, D_IN, D_HID)
    norm, u = _sc1(src, dst, edge_weight, c1)
    v = _prop_second_64(src, dst, b1x, u, norm)
    a2, b2x, c2 = _tc_layer2_head(a1, v, b1.reshape(1, D_HID), wcat2)
    u2 = _prop_first_16(src, dst, c2, norm)
    v2 = _prop_second_16(src, dst, b2x, u2, norm)
    return _tc_final(a2, v2, b2.reshape(1, D_OUT))


# speedup vs baseline: 6.9136x; 6.9136x over previous
import jax
import jax.numpy as jnp
from jax import lax
from jax.experimental import pallas as pl
from jax.experimental.pallas import tpu as pltpu
from jax.experimental.pallas import tpu_sc as plsc

NN = 10000
EE = 320000
D_IN = 128
D_HID = 64
D_OUT = 16
NC = 2
NS = 16
ET = EE // (NC * NS)
ES = EE // NS
CH = 80
RB = 80

_MESH = plsc.VectorSubcoreMesh(core_axis_name="c", subcore_axis_name="s")
_SC_PARAMS = pltpu.CompilerParams(use_tc_tiling_on_sc=False)

_F32 = jnp.float32


def _fill_zero_2d(ref, rows, width):
    def body(i, _):
        for k in range(width // 16):
            ref[i, pl.ds(k * 16, 16)] = jnp.zeros((16,), _F32)
        return 0
    lax.fori_loop(0, rows, body, 0)


def _fill_zero_1d(ref, n):
    def body(i, _):
        ref[pl.ds(i * 16, 16)] = jnp.zeros((16,), _F32)
        return 0
    lax.fori_loop(0, n // 16, body, 0)


def _newton_rsqrt(x):
    i = lax.bitcast_convert_type(x, jnp.int32)
    y = lax.bitcast_convert_type(jnp.int32(0x5F3759DF) - (i >> 1), _F32)
    for _ in range(3):
        y = y * (1.5 - 0.5 * x * y * y)
    return jnp.where(x > 0, y, 0.0)


def _tile_row_hops(s, fn):
    @pl.when(s < 15)
    def _():
        def body(i, _):
            fn(s * 640 + i * RB)
            return 0
        lax.fori_loop(0, 640 // RB, body, 0)

    @pl.when(s == 15)
    def _():
        def body(i, _):
            fn(9600 + i * RB)
            return 0
        lax.fori_loop(0, 400 // RB, body, 0)


def _stage_rows_to_spmem(hbm, sh, bounce, s):
    def hop(r):
        pltpu.sync_copy(hbm.at[pl.ds(r, RB), :], bounce)
        pltpu.sync_copy(bounce, sh.at[pl.ds(r, RB), :])
    _tile_row_hops(s, hop)


def _export_spmem_rows(sh, hbm_out, bounce, c, s):
    def hop(r):
        pltpu.sync_copy(sh.at[pl.ds(r, RB), :], bounce)
        pltpu.sync_copy(bounce, hbm_out.at[c, pl.ds(r, RB), :])
    _tile_row_hops(s, hop)


def _zero_acc_rows(acc_sh, zrows, s):
    def hop(r):
        pltpu.sync_copy(zrows, acc_sh.at[pl.ds(r, RB), :])
    _tile_row_hops(s, hop)


def _scale_rows(rows, norm_v, e0, width):
    nw = width // 16
    for g in range(CH // 16):
        nsv = norm_v[pl.ds(e0 + g * 16, 16)]
        for l in range(16):
            ns = nsv[l]
            r = g * 16 + l
            for k in range(nw):
                rows[r, pl.ds(k * 16, 16)] = rows[r, pl.ds(k * 16, 16)] * ns


def _prop_chunks(src_h, dst_h, norm_h, e_base, nbuf, y_sh, acc_sh,
                 sidx, didx, rows, width):
    def chunk(j, _):
        ge0 = e_base + j * CH
        pltpu.sync_copy(src_h.at[pl.ds(ge0, CH)], sidx)
        pltpu.sync_copy(dst_h.at[pl.ds(ge0, CH)], didx)
        pltpu.sync_copy(norm_h.at[pl.ds(ge0, CH)], nbuf)
        pltpu.sync_copy(y_sh.at[sidx], rows)
        _scale_rows(rows, nbuf, 0, width)
        pltpu.sync_copy(rows, acc_sh.at[didx], add=True)
        return 0
    lax.fori_loop(0, ET // CH, chunk, 0)


def _sc1_body(src_h, dst_h, ew_h, c1_h, norm_h, u_h,
              vbuf, wv, sidx, didx, dsv, ddv, norm_v,
              rows, bounce, zrows, zflat,
              deg_sh, dis_sh, c1_sh, u_sh):
    c = lax.axis_index("c")
    s = lax.axis_index("s")

    _fill_zero_2d(zrows, RB, D_HID)
    _zero_acc_rows(u_sh, zrows, s)
    _stage_rows_to_spmem(c1_h, c1_sh, bounce, s)

    @pl.when(s < 5)
    def _():
        _fill_zero_1d(zflat, 2000)
        pltpu.sync_copy(zflat, deg_sh.at[pl.ds(s * 2000, 2000)])

    plsc.subcore_barrier()

    def deg_chunk(j, _):
        ge0 = s * ES + j * CH
        pltpu.sync_copy(src_h.at[pl.ds(ge0, CH)], sidx)
        pltpu.sync_copy(dst_h.at[pl.ds(ge0, CH)], didx)
        pltpu.sync_copy(ew_h.at[pl.ds(ge0, CH)], wv)
        for k in range(CH // 16):
            d = pl.ds(k * 16, 16)
            vbuf[d] = jnp.where(sidx[d] == didx[d], 0.0, wv[d])
        pltpu.sync_copy(vbuf, deg_sh.at[sidx], add=True)
        return 0
    lax.fori_loop(0, ES // CH, deg_chunk, 0)

    plsc.subcore_barrier()

    def dis_slice(r0, n):
        pltpu.sync_copy(deg_sh.at[pl.ds(r0, n)], norm_v.at[pl.ds(0, n)])

        def body(i, _):
            v = norm_v[pl.ds(i * 16, 16)]
            norm_v[pl.ds(i * 16, 16)] = _newton_rsqrt(v)
            return 0
        lax.fori_loop(0, n // 16, body, 0)
        pltpu.sync_copy(norm_v.at[pl.ds(0, n)], dis_sh.at[pl.ds(r0, n)])

    @pl.when(s < 15)
    def _():
        dis_slice(s * 640, 640)

    @pl.when(s == 15)
    def _():
        dis_slice(9600, 400)

    plsc.subcore_barrier()

    wid = s * NC + c
    e_base = wid * ET

    def chunk(j, _):
        ge0 = e_base + j * CH
        pltpu.sync_copy(src_h.at[pl.ds(ge0, CH)], sidx)
        pltpu.sync_copy(dst_h.at[pl.ds(ge0, CH)], didx)
        pltpu.sync_copy(ew_h.at[pl.ds(ge0, CH)], wv)
        pltpu.sync_copy(dis_sh.at[sidx], dsv)
        pltpu.sync_copy(dis_sh.at[didx], ddv)
        for k in range(CH // 16):
            d = pl.ds(k * 16, 16)
            w = jnp.where(sidx[d] == didx[d], 0.0, wv[d])
            norm_v[pl.ds(j * CH + k * 16, 16)] = -(dsv[d] * w * ddv[d])
        pltpu.sync_copy(c1_sh.at[sidx], rows)
        _scale_rows(rows, norm_v, j * CH, D_HID)
        pltpu.sync_copy(rows, u_sh.at[didx], add=True)
        return 0
    lax.fori_loop(0, ET // CH, chunk, 0)

    pltpu.sync_copy(norm_v, norm_h.at[pl.ds(e_base, ET)])

    plsc.subcore_barrier()
    _export_spmem_rows(u_sh, u_h, bounce, c, s)


def _sc1(src, dst, ew, c1):
    f = pl.kernel(
        _sc1_body,
        out_type=[
            jax.ShapeDtypeStruct((EE,), _F32),
            jax.ShapeDtypeStruct((NC, NN, D_HID), _F32),
        ],
        mesh=_MESH,
        compiler_params=_SC_PARAMS,
        scratch_types=[
            pltpu.VMEM((CH,), _F32),
            pltpu.VMEM((CH,), _F32),
            pltpu.VMEM((CH,), jnp.int32),
            pltpu.VMEM((CH,), jnp.int32),
            pltpu.VMEM((CH,), _F32),
            pltpu.VMEM((CH,), _F32),
            pltpu.VMEM((ET,), _F32),
            pltpu.VMEM((CH, D_HID), _F32),
            pltpu.VMEM((RB, D_HID), _F32),
            pltpu.VMEM((RB, D_HID), _F32),
            pltpu.VMEM((2000,), _F32),
            pltpu.VMEM_SHARED((NN,), _F32),
            pltpu.VMEM_SHARED((NN,), _F32),
            pltpu.VMEM_SHARED((NN, D_HID), _F32),
            pltpu.VMEM_SHARED((NN, D_HID), _F32),
        ],
    )
    return f(src, dst, ew, c1)


def _make_prop_second(width):
    def body(src_h, dst_h, bx_h, u_h, norm_h, v_out,
             sidx, didx, nbuf, rows, bounce, ub0, ub1, zrows,
             z_sh, v_sh):
        c = lax.axis_index("c")
        s = lax.axis_index("s")
        wid = s * NC + c

        _fill_zero_2d(zrows, RB, width)
        _zero_acc_rows(v_sh, zrows, s)

        def zhop(r):
            pltpu.sync_copy(bx_h.at[pl.ds(r, RB), :], bounce)
            pltpu.sync_copy(u_h.at[0, pl.ds(r, RB), :], ub0)
            pltpu.sync_copy(u_h.at[1, pl.ds(r, RB), :], ub1)

            def row(i2, _):
                for k in range(width // 16):
                    d = pl.ds(k * 16, 16)
                    bounce[i2, d] = (bounce[i2, d]
                                     + 2.0 * (ub0[i2, d] + ub1[i2, d]))
                return 0
            lax.fori_loop(0, RB, row, 0)
            pltpu.sync_copy(bounce, z_sh.at[pl.ds(r, RB), :])
        _tile_row_hops(s, zhop)

        plsc.subcore_barrier()
        _prop_chunks(src_h, dst_h, norm_h, wid * ET, nbuf, z_sh, v_sh,
                     sidx, didx, rows, width)
        plsc.subcore_barrier()
        _export_spmem_rows(v_sh, v_out, bounce, c, s)

    def run(src, dst, bx, u, norm):
        f = pl.kernel(
            body,
            out_type=[jax.ShapeDtypeStruct((NC, NN, width), _F32)],
            mesh=_MESH,
            compiler_params=_SC_PARAMS,
            scratch_types=[
                pltpu.VMEM((CH,), jnp.int32),
                pltpu.VMEM((CH,), jnp.int32),
                pltpu.VMEM((CH,), _F32),
                pltpu.VMEM((CH, width), _F32),
                pltpu.VMEM((RB, width), _F32),
                pltpu.VMEM((RB, width), _F32),
                pltpu.VMEM((RB, width), _F32),
                pltpu.VMEM((RB, width), _F32),
                pltpu.VMEM_SHARED((NN, width), _F32),
                pltpu.VMEM_SHARED((NN, width), _F32),
            ],
        )
        return f(src, dst, bx, u, norm)[0]
    return run


def _make_prop_first(width):
    def body(src_h, dst_h, y_h, norm_h, u_out,
             sidx, didx, nbuf, rows, bounce, zrows,
             y_sh, acc_sh):
        c = lax.axis_index("c")
        s = lax.axis_index("s")
        wid = s * NC + c

        _fill_zero_2d(zrows, RB, width)
        _zero_acc_rows(acc_sh, zrows, s)
        _stage_rows_to_spmem(y_h, y_sh, bounce, s)

        plsc.subcore_barrier()
        _prop_chunks(src_h, dst_h, norm_h, wid * ET, nbuf, y_sh, acc_sh,
                     sidx, didx, rows, width)
        plsc.subcore_barrier()
        _export_spmem_rows(acc_sh, u_out, bounce, c, s)

    def run(src, dst, y, norm):
        f = pl.kernel(
            body,
            out_type=[jax.ShapeDtypeStruct((NC, NN, width), _F32)],
            mesh=_MESH,
            compiler_params=_SC_PARAMS,
            scratch_types=[
                pltpu.VMEM((CH,), jnp.int32),
                pltpu.VMEM((CH,), jnp.int32),
                pltpu.VMEM((CH,), _F32),
                pltpu.VMEM((CH, width), _F32),
                pltpu.VMEM((RB, width), _F32),
                pltpu.VMEM((RB, width), _F32),
                pltpu.VMEM_SHARED((NN, width), _F32),
                pltpu.VMEM_SHARED((NN, width), _F32),
            ],
        )
        return f(src, dst, y, norm)[0]
    return run


_prop_second_64 = _make_prop_second(D_HID)
_prop_first_16 = _make_prop_first(D_OUT)
_prop_second_16 = _make_prop_second(D_OUT)


_BN = 2000


def _tc_matmul_split(x, w, d_in, d_out):
    def body(x_ref, w_ref, a_ref, b_ref, c_ref):
        y = jnp.dot(x_ref[...], w_ref[...], preferred_element_type=_F32)
        a_ref[...] = y[:, :d_out]
        b_ref[...] = y[:, d_out:2 * d_out]
        c_ref[...] = y[:, 2 * d_out:]

    out = pl.pallas_call(
        body,
        grid=(NN // _BN,),
        in_specs=[
            pl.BlockSpec((_BN, d_in), lambda i: (i, 0)),
            pl.BlockSpec((d_in, 3 * d_out), lambda i: (0, 0)),
        ],
        out_specs=[pl.BlockSpec((_BN, d_out), lambda i: (i, 0))] * 3,
        out_shape=[jax.ShapeDtypeStruct((NN, d_out), _F32)] * 3,
    )(x, w)
    return out


def _tc_layer2_head(a1, v, b1, w2):
    def body(a_ref, v_ref, b_ref, w_ref, a2_ref, b2_ref, c2_ref):
        h = a_ref[...] + v_ref[0] + v_ref[1] + b_ref[...]
        h = jnp.maximum(h, 0.0)
        z = jnp.dot(h, w_ref[...], preferred_element_type=_F32)
        a2_ref[...] = z[:, :D_OUT]
        b2_ref[...] = z[:, D_OUT:2 * D_OUT]
        c2_ref[...] = z[:, 2 * D_OUT:]

    out = pl.pallas_call(
        body,
        grid=(NN // _BN,),
        in_specs=[
            pl.BlockSpec((_BN, D_HID), lambda i: (i, 0)),
            pl.BlockSpec((NC, _BN, D_HID), lambda i: (0, i, 0)),
            pl.BlockSpec((1, D_HID), lambda i: (0, 0)),
            pl.BlockSpec((D_HID, 3 * D_OUT), lambda i: (0, 0)),
        ],
        out_specs=[pl.BlockSpec((_BN, D_OUT), lambda i: (i, 0))] * 3,
        out_shape=[jax.ShapeDtypeStruct((NN, D_OUT), _F32)] * 3,
    )(a1, v, b1, w2)
    return out


def _tc_final(a2, v2, b2):
    def body(a_ref, v_ref, b_ref, o_ref):
        o_ref[...] = a_ref[...] + v_ref[0] + v_ref[1] + b_ref[...]

    return pl.pallas_call(
        body,
        grid=(NN // _BN,),
        in_specs=[
            pl.BlockSpec((_BN, D_OUT), lambda i: (i, 0)),
            pl.BlockSpec((NC, _BN, D_OUT), lambda i: (0, i, 0)),
            pl.BlockSpec((1, D_OUT), lambda i: (0, 0)),
        ],
        out_specs=pl.BlockSpec((_BN, D_OUT), lambda i: (i, 0)),
        out_shape=jax.ShapeDtypeStruct((NN, D_OUT), _F32),
    )(a2, v2, b2)


def kernel(x, edge_index, edge_weight, W1, b1, W2, b2):
    src = edge_index[0]
    dst = edge_index[1]

    w1a, w1b, w1c = W1[:D_IN], W1[D_IN:2 * D_IN], W1[2 * D_IN:]
    wcat1 = jnp.concatenate([w1a - w1c, w1b, w1c], axis=1)
    w2a, w2b, w2c = W2[:D_HID], W2[D_HID:2 * D_HID], W2[2 * D_HID:]
    wcat2 = jnp.concatenate([w2a - w2c, w2b, w2c], axis=1)

    a1, b1x, c1 = _tc_matmul_split(x, wcat1, D_IN, D_HID)
    norm, u = _sc1(src, dst, edge_weight, c1)
    v = _prop_second_64(src, dst, b1x, u, norm)
    a2, b2x, c2 = _tc_layer2_head(a1, v, b1.reshape(1, D_HID), wcat2)
    u2 = _prop_first_16(src, dst, c2, norm)
    v2 = _prop_second_16(src, dst, b2x, u2, norm)
    return _tc_final(a2, v2, b2.reshape(1, D_OUT))


# speedup vs baseline: 16.3339x; 2.3626x over previous
import jax
import jax.numpy as jnp
from jax import lax
from jax.experimental import pallas as pl
from jax.experimental.pallas import tpu as pltpu
from jax.experimental.pallas import tpu_sc as plsc

NN = 10000
EE = 320000
D_IN = 128
D_HID = 64
D_OUT = 16
NC = 2
NS = 16
ET = EE // (NC * NS)
ES = EE // NS
CH = 80
EB = 2000
RB = 80

_MESH = plsc.VectorSubcoreMesh(core_axis_name="c", subcore_axis_name="s")
_SC_PARAMS = pltpu.CompilerParams(use_tc_tiling_on_sc=False)

_F32 = jnp.float32


def _fill_zero_2d(ref, rows, width):
    def body(i, _):
        for k in range(width // 16):
            ref[i, pl.ds(k * 16, 16)] = jnp.zeros((16,), _F32)
        return 0
    lax.fori_loop(0, rows, body, 0)


def _fill_zero_1d(ref, n):
    def body(i, _):
        ref[pl.ds(i * 16, 16)] = jnp.zeros((16,), _F32)
        return 0
    lax.fori_loop(0, n // 16, body, 0)


def _newton_rsqrt(x):
    i = lax.bitcast_convert_type(x, jnp.int32)
    y = lax.bitcast_convert_type(jnp.int32(0x5F3759DF) - (i >> 1), _F32)
    for _ in range(3):
        y = y * (1.5 - 0.5 * x * y * y)
    return jnp.where(x > 0, y, 0.0)


def _tile_row_hops(s, fn):
    @pl.when(s < 15)
    def _():
        def body(i, _):
            fn(s * 640 + i * RB)
            return 0
        lax.fori_loop(0, 640 // RB, body, 0)

    @pl.when(s == 15)
    def _():
        def body(i, _):
            fn(9600 + i * RB)
            return 0
        lax.fori_loop(0, 400 // RB, body, 0)


def _stage_rows_to_spmem(hbm, sh, bounce, s):
    def hop(r):
        pltpu.sync_copy(hbm.at[pl.ds(r, RB), :], bounce)
        pltpu.sync_copy(bounce, sh.at[pl.ds(r, RB), :])
    _tile_row_hops(s, hop)


def _export_spmem_rows(sh, hbm_out, bounce, c, s):
    def hop(r):
        pltpu.sync_copy(sh.at[pl.ds(r, RB), :], bounce)
        pltpu.sync_copy(bounce, hbm_out.at[c, pl.ds(r, RB), :])
    _tile_row_hops(s, hop)


def _zero_acc_rows(acc_sh, zrows, s):
    def hop(r):
        pltpu.sync_copy(zrows, acc_sh.at[pl.ds(r, RB), :])
    _tile_row_hops(s, hop)


def _scale_rows(rows, norm_v, e0, width):
    nw = width // 16
    for g in range(CH // 16):
        nsv = norm_v[pl.ds(e0 + g * 16, 16)]
        for l in range(16):
            ns = nsv[l]
            r = g * 16 + l
            for k in range(nw):
                rows[r, pl.ds(k * 16, 16)] = rows[r, pl.ds(k * 16, 16)] * ns


def _copy_idx(dstbuf, srcbuf, off):
    for k in range(CH // 16):
        dstbuf[pl.ds(k * 16, 16)] = srcbuf[pl.ds(off + k * 16, 16)]


def _prop_chunks(src_h, dst_h, norm_h, e_base, s_blk, d_blk, n_blk, didx,
                 y_sh, acc_sh, rows, width):
    def block(b, _):
        ge0 = e_base + b * EB
        pltpu.sync_copy(src_h.at[pl.ds(ge0, EB)], s_blk)
        pltpu.sync_copy(dst_h.at[pl.ds(ge0, EB)], d_blk)
        pltpu.sync_copy(norm_h.at[pl.ds(ge0, EB)], n_blk)

        def chunk(j, _):
            o = j * CH
            _copy_idx(didx, d_blk, o)
            pltpu.sync_copy(y_sh.at[s_blk.at[pl.ds(o, CH)]], rows)
            _scale_rows(rows, n_blk, o, width)
            pltpu.sync_copy(rows, acc_sh.at[didx], add=True)
            return 0
        lax.fori_loop(0, EB // CH, chunk, 0)
        return 0
    lax.fori_loop(0, ET // EB, block, 0)


def _sc1_body(src_h, dst_h, ew_h, c1_h, norm_h, u_h,
              s_blk, d_blk, w_blk, vbuf, sidx, didx, dsv, ddv, norm_v,
              rows, bounce, zrows, zflat,
              deg_sh, dis_sh, c1_sh, u_sh):
    c = lax.axis_index("c")
    s = lax.axis_index("s")

    _fill_zero_2d(zrows, RB, D_HID)
    _zero_acc_rows(u_sh, zrows, s)
    _stage_rows_to_spmem(c1_h, c1_sh, bounce, s)

    @pl.when(s < 5)
    def _():
        _fill_zero_1d(zflat, 2000)
        pltpu.sync_copy(zflat, deg_sh.at[pl.ds(s * 2000, 2000)])

    plsc.subcore_barrier()

    def deg_block(b, _):
        ge0 = s * ES + b * EB
        pltpu.sync_copy(src_h.at[pl.ds(ge0, EB)], s_blk)
        pltpu.sync_copy(dst_h.at[pl.ds(ge0, EB)], d_blk)
        pltpu.sync_copy(ew_h.at[pl.ds(ge0, EB)], w_blk)

        def chunk(j, _):
            o = j * CH
            _copy_idx(sidx, s_blk, o)
            for k in range(CH // 16):
                d = pl.ds(k * 16, 16)
                do = pl.ds(o + k * 16, 16)
                vbuf[d] = jnp.where(s_blk[do] == d_blk[do], 0.0, w_blk[do])
            pltpu.sync_copy(vbuf, deg_sh.at[sidx], add=True)
            return 0
        lax.fori_loop(0, EB // CH, chunk, 0)
        return 0
    lax.fori_loop(0, ES // EB, deg_block, 0)

    plsc.subcore_barrier()

    def dis_slice(r0, n):
        pltpu.sync_copy(deg_sh.at[pl.ds(r0, n)], norm_v.at[pl.ds(0, n)])

        def body(i, _):
            v = norm_v[pl.ds(i * 16, 16)]
            norm_v[pl.ds(i * 16, 16)] = _newton_rsqrt(v)
            return 0
        lax.fori_loop(0, n // 16, body, 0)
        pltpu.sync_copy(norm_v.at[pl.ds(0, n)], dis_sh.at[pl.ds(r0, n)])

    @pl.when(s < 15)
    def _():
        dis_slice(s * 640, 640)

    @pl.when(s == 15)
    def _():
        dis_slice(9600, 400)

    plsc.subcore_barrier()

    wid = s * NC + c
    e_base = wid * ET

    def np_block(b, _):
        ge0 = e_base + b * EB
        pltpu.sync_copy(src_h.at[pl.ds(ge0, EB)], s_blk)
        pltpu.sync_copy(dst_h.at[pl.ds(ge0, EB)], d_blk)
        pltpu.sync_copy(ew_h.at[pl.ds(ge0, EB)], w_blk)

        def chunk(j, _):
            o = j * CH
            _copy_idx(didx, d_blk, o)
            pltpu.sync_copy(dis_sh.at[s_blk.at[pl.ds(o, CH)]], dsv)
            pltpu.sync_copy(dis_sh.at[d_blk.at[pl.ds(o, CH)]], ddv)
            for k in range(CH // 16):
                d = pl.ds(k * 16, 16)
                do = pl.ds(o + k * 16, 16)
                w = jnp.where(s_blk[do] == d_blk[do], 0.0, w_blk[do])
                norm_v[pl.ds(b * EB + o + k * 16, 16)] = -(dsv[d] * w * ddv[d])
            pltpu.sync_copy(c1_sh.at[s_blk.at[pl.ds(o, CH)]], rows)
            _scale_rows(rows, norm_v, b * EB + o, D_HID)
            pltpu.sync_copy(rows, u_sh.at[didx], add=True)
            return 0
        lax.fori_loop(0, EB // CH, chunk, 0)
        return 0
    lax.fori_loop(0, ET // EB, np_block, 0)

    pltpu.sync_copy(norm_v, norm_h.at[pl.ds(e_base, ET)])

    plsc.subcore_barrier()
    _export_spmem_rows(u_sh, u_h, bounce, c, s)


def _sc1(src, dst, ew, c1):
    f = pl.kernel(
        _sc1_body,
        out_type=[
            jax.ShapeDtypeStruct((EE,), _F32),
            jax.ShapeDtypeStruct((NC, NN, D_HID), _F32),
        ],
        mesh=_MESH,
        compiler_params=_SC_PARAMS,
        scratch_types=[
            pltpu.VMEM((EB,), jnp.int32),
            pltpu.VMEM((EB,), jnp.int32),
            pltpu.VMEM((EB,), _F32),
            pltpu.VMEM((CH,), _F32),
            pltpu.VMEM((CH,), jnp.int32),
            pltpu.VMEM((CH,), jnp.int32),
            pltpu.VMEM((CH,), _F32),
            pltpu.VMEM((CH,), _F32),
            pltpu.VMEM((ET,), _F32),
            pltpu.VMEM((CH, D_HID), _F32),
            pltpu.VMEM((RB, D_HID), _F32),
            pltpu.VMEM((RB, D_HID), _F32),
            pltpu.VMEM((2000,), _F32),
            pltpu.VMEM_SHARED((NN,), _F32),
            pltpu.VMEM_SHARED((NN,), _F32),
            pltpu.VMEM_SHARED((NN, D_HID), _F32),
            pltpu.VMEM_SHARED((NN, D_HID), _F32),
        ],
    )
    return f(src, dst, ew, c1)


def _make_prop_second(width):
    def body(src_h, dst_h, bx_h, u_h, norm_h, v_out,
             s_blk, d_blk, n_blk, didx, rows, bounce, ub0, ub1, zrows,
             z_sh, v_sh):
        c = lax.axis_index("c")
        s = lax.axis_index("s")
        wid = s * NC + c

        _fill_zero_2d(zrows, RB, width)
        _zero_acc_rows(v_sh, zrows, s)

        def zhop(r):
            pltpu.sync_copy(bx_h.at[pl.ds(r, RB), :], bounce)
            pltpu.sync_copy(u_h.at[0, pl.ds(r, RB), :], ub0)
            pltpu.sync_copy(u_h.at[1, pl.ds(r, RB), :], ub1)

            def row(i2, _):
                for k in range(width // 16):
                    d = pl.ds(k * 16, 16)
                    bounce[i2, d] = (bounce[i2, d]
                                     + 2.0 * (ub0[i2, d] + ub1[i2, d]))
                return 0
            lax.fori_loop(0, RB, row, 0)
            pltpu.sync_copy(bounce, z_sh.at[pl.ds(r, RB), :])
        _tile_row_hops(s, zhop)

        plsc.subcore_barrier()
        _prop_chunks(src_h, dst_h, norm_h, wid * ET, s_blk, d_blk, n_blk,
                     didx, z_sh, v_sh, rows, width)
        plsc.subcore_barrier()
        _export_spmem_rows(v_sh, v_out, bounce, c, s)

    def run(src, dst, bx, u, norm):
        f = pl.kernel(
            body,
            out_type=[jax.ShapeDtypeStruct((NC, NN, width), _F32)],
            mesh=_MESH,
            compiler_params=_SC_PARAMS,
            scratch_types=[
                pltpu.VMEM((EB,), jnp.int32),
                pltpu.VMEM((EB,), jnp.int32),
                pltpu.VMEM((EB,), _F32),
                pltpu.VMEM((CH,), jnp.int32),
                pltpu.VMEM((CH, width), _F32),
                pltpu.VMEM((RB, width), _F32),
                pltpu.VMEM((RB, width), _F32),
                pltpu.VMEM((RB, width), _F32),
                pltpu.VMEM((RB, width), _F32),
                pltpu.VMEM_SHARED((NN, width), _F32),
                pltpu.VMEM_SHARED((NN, width), _F32),
            ],
        )
        return f(src, dst, bx, u, norm)[0]
    return run


def _make_prop_first(width):
    def body(src_h, dst_h, y_h, norm_h, u_out,
             s_blk, d_blk, n_blk, didx, rows, bounce, zrows,
             y_sh, acc_sh):
        c = lax.axis_index("c")
        s = lax.axis_index("s")
        wid = s * NC + c

        _fill_zero_2d(zrows, RB, width)
        _zero_acc_rows(acc_sh, zrows, s)
        _stage_rows_to_spmem(y_h, y_sh, bounce, s)

        plsc.subcore_barrier()
        _prop_chunks(src_h, dst_h, norm_h, wid * ET, s_blk, d_blk, n_blk,
                     didx, y_sh, acc_sh, rows, width)
        plsc.subcore_barrier()
        _export_spmem_rows(acc_sh, u_out, bounce, c, s)

    def run(src, dst, y, norm):
        f = pl.kernel(
            body,
            out_type=[jax.ShapeDtypeStruct((NC, NN, width), _F32)],
            mesh=_MESH,
            compiler_params=_SC_PARAMS,
            scratch_types=[
                pltpu.VMEM((EB,), jnp.int32),
                pltpu.VMEM((EB,), jnp.int32),
                pltpu.VMEM((EB,), _F32),
                pltpu.VMEM((CH,), jnp.int32),
                pltpu.VMEM((CH, width), _F32),
                pltpu.VMEM((RB, width), _F32),
                pltpu.VMEM((RB, width), _F32),
                pltpu.VMEM_SHARED((NN, width), _F32),
                pltpu.VMEM_SHARED((NN, width), _F32),
            ],
        )
        return f(src, dst, y, norm)[0]
    return run


_prop_second_64 = _make_prop_second(D_HID)
_prop_first_16 = _make_prop_first(D_OUT)
_prop_second_16 = _make_prop_second(D_OUT)


_BN = 2000


def _tc_matmul_split(x, w, d_in, d_out):
    def body(x_ref, w_ref, a_ref, b_ref, c_ref):
        y = jnp.dot(x_ref[...], w_ref[...], preferred_element_type=_F32)
        a_ref[...] = y[:, :d_out]
        b_ref[...] = y[:, d_out:2 * d_out]
        c_ref[...] = y[:, 2 * d_out:]

    out = pl.pallas_call(
        body,
        grid=(NN // _BN,),
        in_specs=[
            pl.BlockSpec((_BN, d_in), lambda i: (i, 0)),
            pl.BlockSpec((d_in, 3 * d_out), lambda i: (0, 0)),
        ],
        out_specs=[pl.BlockSpec((_BN, d_out), lambda i: (i, 0))] * 3,
        out_shape=[jax.ShapeDtypeStruct((NN, d_out), _F32)] * 3,
    )(x, w)
    return out


def _tc_layer2_head(a1, v, b1, w2):
    def body(a_ref, v_ref, b_ref, w_ref, a2_ref, b2_ref, c2_ref):
        h = a_ref[...] + v_ref[0] + v_ref[1] + b_ref[...]
        h = jnp.maximum(h, 0.0)
        z = jnp.dot(h, w_ref[...], preferred_element_type=_F32)
        a2_ref[...] = z[:, :D_OUT]
        b2_ref[...] = z[:, D_OUT:2 * D_OUT]
        c2_ref[...] = z[:, 2 * D_OUT:]

    out = pl.pallas_call(
        body,
        grid=(NN // _BN,),
        in_specs=[
            pl.BlockSpec((_BN, D_HID), lambda i: (i, 0)),
            pl.BlockSpec((NC, _BN, D_HID), lambda i: (0, i, 0)),
            pl.BlockSpec((1, D_HID), lambda i: (0, 0)),
            pl.BlockSpec((D_HID, 3 * D_OUT), lambda i: (0, 0)),
        ],
        out_specs=[pl.BlockSpec((_BN, D_OUT), lambda i: (i, 0))] * 3,
        out_shape=[jax.ShapeDtypeStruct((NN, D_OUT), _F32)] * 3,
    )(a1, v, b1, w2)
    return out


def _tc_final(a2, v2, b2):
    def body(a_ref, v_ref, b_ref, o_ref):
        o_ref[...] = a_ref[...] + v_ref[0] + v_ref[1] + b_ref[...]

    return pl.pallas_call(
        body,
        grid=(NN // _BN,),
        in_specs=[
            pl.BlockSpec((_BN, D_OUT), lambda i: (i, 0)),
            pl.BlockSpec((NC, _BN, D_OUT), lambda i: (0, i, 0)),
            pl.BlockSpec((1, D_OUT), lambda i: (0, 0)),
        ],
        out_specs=pl.BlockSpec((_BN, D_OUT), lambda i: (i, 0)),
        out_shape=jax.ShapeDtypeStruct((NN, D_OUT), _F32),
    )(a2, v2, b2)


def kernel(x, edge_index, edge_weight, W1, b1, W2, b2):
    src = edge_index[0]
    dst = edge_index[1]

    w1a, w1b, w1c = W1[:D_IN], W1[D_IN:2 * D_IN], W1[2 * D_IN:]
    wcat1 = jnp.concatenate([w1a - w1c, w1b, w1c], axis=1)
    w2a, w2b, w2c = W2[:D_HID], W2[D_HID:2 * D_HID], W2[2 * D_HID:]
    wcat2 = jnp.concatenate([w2a - w2c, w2b, w2c], axis=1)

    a1, b1x, c1 = _tc_matmul_split(x, wcat1, D_IN, D_HID)
    norm, u = _sc1(src, dst, edge_weight, c1)
    v = _prop_second_64(src, dst, b1x, u, norm)
    a2, b2x, c2 = _tc_layer2_head(a1, v, b1.reshape(1, D_HID), wcat2)
    u2 = _prop_first_16(src, dst, c2, norm)
    v2 = _prop_second_16(src, dst, b2x, u2, norm)
    return _tc_final(a2, v2, b2.reshape(1, D_OUT))


# speedup vs baseline: 18.1226x; 1.1095x over previous
import jax
import jax.numpy as jnp
from jax import lax
from jax.experimental import pallas as pl
from jax.experimental.pallas import tpu as pltpu
from jax.experimental.pallas import tpu_sc as plsc

NN = 10000
EE = 320000
D_IN = 128
D_HID = 64
D_OUT = 16
NC = 2
NS = 16
ET = EE // (NC * NS)
ES = EE // NS
CH = 80
EB = 2000
RB = 80

_MESH = plsc.VectorSubcoreMesh(core_axis_name="c", subcore_axis_name="s")
_SC_PARAMS = pltpu.CompilerParams(use_tc_tiling_on_sc=False)

_F32 = jnp.float32


def _fill_zero_2d(ref, rows, width):
    def body(i, _):
        for k in range(width // 16):
            ref[i, pl.ds(k * 16, 16)] = jnp.zeros((16,), _F32)
        return 0
    lax.fori_loop(0, rows, body, 0)


def _fill_zero_1d(ref, n):
    def body(i, _):
        ref[pl.ds(i * 16, 16)] = jnp.zeros((16,), _F32)
        return 0
    lax.fori_loop(0, n // 16, body, 0)


def _newton_rsqrt(x):
    i = lax.bitcast_convert_type(x, jnp.int32)
    y = lax.bitcast_convert_type(jnp.int32(0x5F3759DF) - (i >> 1), _F32)
    for _ in range(3):
        y = y * (1.5 - 0.5 * x * y * y)
    return jnp.where(x > 0, y, 0.0)


def _tile_row_hops(s, fn):
    @pl.when(s < 15)
    def _():
        def body(i, _):
            fn(s * 640 + i * RB)
            return 0
        lax.fori_loop(0, 640 // RB, body, 0)

    @pl.when(s == 15)
    def _():
        def body(i, _):
            fn(9600 + i * RB)
            return 0
        lax.fori_loop(0, 400 // RB, body, 0)


def _stage_rows_to_spmem(hbm, sh, bounce, s):
    def hop(r):
        pltpu.sync_copy(hbm.at[pl.ds(r, RB), :], bounce)
        pltpu.sync_copy(bounce, sh.at[pl.ds(r, RB), :])
    _tile_row_hops(s, hop)


def _export_spmem_rows(sh, hbm_out, bounce, c, s):
    def hop(r):
        pltpu.sync_copy(sh.at[pl.ds(r, RB), :], bounce)
        pltpu.sync_copy(bounce, hbm_out.at[c, pl.ds(r, RB), :])
    _tile_row_hops(s, hop)


def _zero_acc_rows(acc_sh, zrows, s):
    def hop(r):
        pltpu.sync_copy(zrows, acc_sh.at[pl.ds(r, RB), :])
    _tile_row_hops(s, hop)


def _scale_rows(rows, norm_v, e0, width):
    nw = width // 16
    for g in range(CH // 16):
        nsv = norm_v[pl.ds(e0 + g * 16, 16)]
        for l in range(16):
            ns = nsv[l]
            r = g * 16 + l
            for k in range(nw):
                rows[r, pl.ds(k * 16, 16)] = rows[r, pl.ds(k * 16, 16)] * ns


def _copy_idx(dstbuf, srcbuf, off):
    for k in range(CH // 16):
        dstbuf[pl.ds(k * 16, 16)] = srcbuf[pl.ds(off + k * 16, 16)]


def _prop_chunks(src_h, dst_h, norm_h, e_base, s_blk, d_blk, n_blk,
                 didx0, didx1, rows0, rows1, gsem0, gsem1,
                 y_sh, acc_sh, width):
    def g_start(o, rows, sem):
        pltpu.async_copy(y_sh.at[s_blk.at[pl.ds(o, CH)]], rows, sem)

    def g_wait(o, rows, sem):
        pltpu.make_async_copy(y_sh.at[s_blk.at[pl.ds(o, CH)]], rows,
                              sem).wait()

    def consume(o, rows, didx):
        _copy_idx(didx, d_blk, o)
        _scale_rows(rows, n_blk, o, width)
        pltpu.sync_copy(rows, acc_sh.at[didx], add=True)

    def block(b, _):
        ge0 = e_base + b * EB
        pltpu.sync_copy(src_h.at[pl.ds(ge0, EB)], s_blk)
        pltpu.sync_copy(dst_h.at[pl.ds(ge0, EB)], d_blk)
        pltpu.sync_copy(norm_h.at[pl.ds(ge0, EB)], n_blk)
        g_start(0, rows0, gsem0)

        def pair(p, _):
            o_a = 2 * p * CH
            o_b = o_a + CH
            o_c = o_b + CH
            g_start(o_b, rows1, gsem1)
            g_wait(o_a, rows0, gsem0)
            consume(o_a, rows0, didx0)
            g_start(o_c, rows0, gsem0)
            g_wait(o_b, rows1, gsem1)
            consume(o_b, rows1, didx1)
            return 0
        lax.fori_loop(0, (EB // CH) // 2, pair, 0)

        o_t = (EB // CH - 1) * CH
        g_wait(o_t, rows0, gsem0)
        consume(o_t, rows0, didx0)
        return 0
    lax.fori_loop(0, ET // EB, block, 0)


def _sc1_body(src_h, dst_h, ew_h, c1_h, norm_h, u_h,
              s_blk, d_blk, w_blk, vbuf, sidx, didx, dsv, ddv, norm_v,
              rows0, rows1, gsem0, gsem1, bounce, zrows, zflat,
              deg_sh, dis_sh, c1_sh, u_sh):
    c = lax.axis_index("c")
    s = lax.axis_index("s")

    _fill_zero_2d(zrows, RB, D_HID)
    _zero_acc_rows(u_sh, zrows, s)
    _stage_rows_to_spmem(c1_h, c1_sh, bounce, s)

    @pl.when(s < 5)
    def _():
        _fill_zero_1d(zflat, 2000)
        pltpu.sync_copy(zflat, deg_sh.at[pl.ds(s * 2000, 2000)])

    plsc.subcore_barrier()

    def deg_block(b, _):
        ge0 = s * ES + b * EB
        pltpu.sync_copy(src_h.at[pl.ds(ge0, EB)], s_blk)
        pltpu.sync_copy(dst_h.at[pl.ds(ge0, EB)], d_blk)
        pltpu.sync_copy(ew_h.at[pl.ds(ge0, EB)], w_blk)

        def chunk(j, _):
            o = j * CH
            _copy_idx(sidx, s_blk, o)
            for k in range(CH // 16):
                d = pl.ds(k * 16, 16)
                do = pl.ds(o + k * 16, 16)
                vbuf[d] = jnp.where(s_blk[do] == d_blk[do], 0.0, w_blk[do])
            pltpu.sync_copy(vbuf, deg_sh.at[sidx], add=True)
            return 0
        lax.fori_loop(0, EB // CH, chunk, 0)
        return 0
    lax.fori_loop(0, ES // EB, deg_block, 0)

    plsc.subcore_barrier()

    def dis_slice(r0, n):
        pltpu.sync_copy(deg_sh.at[pl.ds(r0, n)], norm_v.at[pl.ds(0, n)])

        def body(i, _):
            v = norm_v[pl.ds(i * 16, 16)]
            norm_v[pl.ds(i * 16, 16)] = _newton_rsqrt(v)
            return 0
        lax.fori_loop(0, n // 16, body, 0)
        pltpu.sync_copy(norm_v.at[pl.ds(0, n)], dis_sh.at[pl.ds(r0, n)])

    @pl.when(s < 15)
    def _():
        dis_slice(s * 640, 640)

    @pl.when(s == 15)
    def _():
        dis_slice(9600, 400)

    plsc.subcore_barrier()

    wid = s * NC + c
    e_base = wid * ET

    def norm_block(b, _):
        ge0 = e_base + b * EB
        pltpu.sync_copy(src_h.at[pl.ds(ge0, EB)], s_blk)
        pltpu.sync_copy(dst_h.at[pl.ds(ge0, EB)], d_blk)
        pltpu.sync_copy(ew_h.at[pl.ds(ge0, EB)], w_blk)

        def chunk(j, _):
            o = j * CH
            pltpu.sync_copy(dis_sh.at[s_blk.at[pl.ds(o, CH)]], dsv)
            pltpu.sync_copy(dis_sh.at[d_blk.at[pl.ds(o, CH)]], ddv)
            for k in range(CH // 16):
                d = pl.ds(k * 16, 16)
                do = pl.ds(o + k * 16, 16)
                w = jnp.where(s_blk[do] == d_blk[do], 0.0, w_blk[do])
                norm_v[pl.ds(b * EB + o + k * 16, 16)] = -(dsv[d] * w * ddv[d])
            return 0
        lax.fori_loop(0, EB // CH, chunk, 0)
        return 0
    lax.fori_loop(0, ET // EB, norm_block, 0)

    pltpu.sync_copy(norm_v, norm_h.at[pl.ds(e_base, ET)])

    _prop_chunks(src_h, dst_h, norm_h, e_base, s_blk, d_blk, w_blk,
                 sidx, didx, rows0, rows1, gsem0, gsem1,
                 c1_sh, u_sh, D_HID)

    plsc.subcore_barrier()
    _export_spmem_rows(u_sh, u_h, bounce, c, s)


def _sc1(src, dst, ew, c1):
    f = pl.kernel(
        _sc1_body,
        out_type=[
            jax.ShapeDtypeStruct((EE,), _F32),
            jax.ShapeDtypeStruct((NC, NN, D_HID), _F32),
        ],
        mesh=_MESH,
        compiler_params=_SC_PARAMS,
        scratch_types=[
            pltpu.VMEM((EB,), jnp.int32),
            pltpu.VMEM((EB,), jnp.int32),
            pltpu.VMEM((EB,), _F32),
            pltpu.VMEM((CH,), _F32),
            pltpu.VMEM((CH,), jnp.int32),
            pltpu.VMEM((CH,), jnp.int32),
            pltpu.VMEM((CH,), _F32),
            pltpu.VMEM((CH,), _F32),
            pltpu.VMEM((ET,), _F32),
            pltpu.VMEM((CH, D_HID), _F32),
            pltpu.VMEM((CH, D_HID), _F32),
            pltpu.SemaphoreType.DMA,
            pltpu.SemaphoreType.DMA,
            pltpu.VMEM((RB, D_HID), _F32),
            pltpu.VMEM((RB, D_HID), _F32),
            pltpu.VMEM((2000,), _F32),
            pltpu.VMEM_SHARED((NN,), _F32),
            pltpu.VMEM_SHARED((NN,), _F32),
            pltpu.VMEM_SHARED((NN, D_HID), _F32),
            pltpu.VMEM_SHARED((NN, D_HID), _F32),
        ],
    )
    return f(src, dst, ew, c1)


def _make_prop_second(width):
    def body(src_h, dst_h, bx_h, u_h, norm_h, v_out,
             s_blk, d_blk, n_blk, didx0, didx1, rows0, rows1, gsem0, gsem1,
             bounce, ub0, ub1, zrows, z_sh, v_sh):
        c = lax.axis_index("c")
        s = lax.axis_index("s")
        wid = s * NC + c

        _fill_zero_2d(zrows, RB, width)
        _zero_acc_rows(v_sh, zrows, s)

        def zhop(r):
            pltpu.sync_copy(bx_h.at[pl.ds(r, RB), :], bounce)
            pltpu.sync_copy(u_h.at[0, pl.ds(r, RB), :], ub0)
            pltpu.sync_copy(u_h.at[1, pl.ds(r, RB), :], ub1)

            def row(i2, _):
                for k in range(width // 16):
                    d = pl.ds(k * 16, 16)
                    bounce[i2, d] = (bounce[i2, d]
                                     + 2.0 * (ub0[i2, d] + ub1[i2, d]))
                return 0
            lax.fori_loop(0, RB, row, 0)
            pltpu.sync_copy(bounce, z_sh.at[pl.ds(r, RB), :])
        _tile_row_hops(s, zhop)

        plsc.subcore_barrier()
        _prop_chunks(src_h, dst_h, norm_h, wid * ET, s_blk, d_blk, n_blk,
                     didx0, didx1, rows0, rows1, gsem0, gsem1,
                     z_sh, v_sh, width)
        plsc.subcore_barrier()
        _export_spmem_rows(v_sh, v_out, bounce, c, s)

    def run(src, dst, bx, u, norm):
        f = pl.kernel(
            body,
            out_type=[jax.ShapeDtypeStruct((NC, NN, width), _F32)],
            mesh=_MESH,
            compiler_params=_SC_PARAMS,
            scratch_types=[
                pltpu.VMEM((EB,), jnp.int32),
                pltpu.VMEM((EB,), jnp.int32),
                pltpu.VMEM((EB,), _F32),
                pltpu.VMEM((CH,), jnp.int32),
                pltpu.VMEM((CH,), jnp.int32),
                pltpu.VMEM((CH, width), _F32),
                pltpu.VMEM((CH, width), _F32),
                pltpu.SemaphoreType.DMA,
                pltpu.SemaphoreType.DMA,
                pltpu.VMEM((RB, width), _F32),
                pltpu.VMEM((RB, width), _F32),
                pltpu.VMEM((RB, width), _F32),
                pltpu.VMEM((RB, width), _F32),
                pltpu.VMEM_SHARED((NN, width), _F32),
                pltpu.VMEM_SHARED((NN, width), _F32),
            ],
        )
        return f(src, dst, bx, u, norm)[0]
    return run


def _make_prop_first(width):
    def body(src_h, dst_h, y_h, norm_h, u_out,
             s_blk, d_blk, n_blk, didx0, didx1, rows0, rows1, gsem0, gsem1,
             bounce, zrows, y_sh, acc_sh):
        c = lax.axis_index("c")
        s = lax.axis_index("s")
        wid = s * NC + c

        _fill_zero_2d(zrows, RB, width)
        _zero_acc_rows(acc_sh, zrows, s)
        _stage_rows_to_spmem(y_h, y_sh, bounce, s)

        plsc.subcore_barrier()
        _prop_chunks(src_h, dst_h, norm_h, wid * ET, s_blk, d_blk, n_blk,
                     didx0, didx1, rows0, rows1, gsem0, gsem1,
                     y_sh, acc_sh, width)
        plsc.subcore_barrier()
        _export_spmem_rows(acc_sh, u_out, bounce, c, s)

    def run(src, dst, y, norm):
        f = pl.kernel(
            body,
            out_type=[jax.ShapeDtypeStruct((NC, NN, width), _F32)],
            mesh=_MESH,
            compiler_params=_SC_PARAMS,
            scratch_types=[
                pltpu.VMEM((EB,), jnp.int32),
                pltpu.VMEM((EB,), jnp.int32),
                pltpu.VMEM((EB,), _F32),
                pltpu.VMEM((CH,), jnp.int32),
                pltpu.VMEM((CH,), jnp.int32),
                pltpu.VMEM((CH, width), _F32),
                pltpu.VMEM((CH, width), _F32),
                pltpu.SemaphoreType.DMA,
                pltpu.SemaphoreType.DMA,
                pltpu.VMEM((RB, width), _F32),
                pltpu.VMEM((RB, width), _F32),
                pltpu.VMEM_SHARED((NN, width), _F32),
                pltpu.VMEM_SHARED((NN, width), _F32),
            ],
        )
        return f(src, dst, y, norm)[0]
    return run


_prop_second_64 = _make_prop_second(D_HID)
_prop_first_16 = _make_prop_first(D_OUT)
_prop_second_16 = _make_prop_second(D_OUT)


_BN = 2000


def _tc_matmul_split(x, w, d_in, d_out):
    def body(x_ref, w_ref, a_ref, b_ref, c_ref):
        y = jnp.dot(x_ref[...], w_ref[...], preferred_element_type=_F32)
        a_ref[...] = y[:, :d_out]
        b_ref[...] = y[:, d_out:2 * d_out]
        c_ref[...] = y[:, 2 * d_out:]

    out = pl.pallas_call(
        body,
        grid=(NN // _BN,),
        in_specs=[
            pl.BlockSpec((_BN, d_in), lambda i: (i, 0)),
            pl.BlockSpec((d_in, 3 * d_out), lambda i: (0, 0)),
        ],
        out_specs=[pl.BlockSpec((_BN, d_out), lambda i: (i, 0))] * 3,
        out_shape=[jax.ShapeDtypeStruct((NN, d_out), _F32)] * 3,
    )(x, w)
    return out


def _tc_layer2_head(a1, v, b1, w2):
    def body(a_ref, v_ref, b_ref, w_ref, a2_ref, b2_ref, c2_ref):
        h = a_ref[...] + v_ref[0] + v_ref[1] + b_ref[...]
        h = jnp.maximum(h, 0.0)
        z = jnp.dot(h, w_ref[...], preferred_element_type=_F32)
        a2_ref[...] = z[:, :D_OUT]
        b2_ref[...] = z[:, D_OUT:2 * D_OUT]
        c2_ref[...] = z[:, 2 * D_OUT:]

    out = pl.pallas_call(
        body,
        grid=(NN // _BN,),
        in_specs=[
            pl.BlockSpec((_BN, D_HID), lambda i: (i, 0)),
            pl.BlockSpec((NC, _BN, D_HID), lambda i: (0, i, 0)),
            pl.BlockSpec((1, D_HID), lambda i: (0, 0)),
            pl.BlockSpec((D_HID, 3 * D_OUT), lambda i: (0, 0)),
        ],
        out_specs=[pl.BlockSpec((_BN, D_OUT), lambda i: (i, 0))] * 3,
        out_shape=[jax.ShapeDtypeStruct((NN, D_OUT), _F32)] * 3,
    )(a1, v, b1, w2)
    return out


def _tc_final(a2, v2, b2):
    def body(a_ref, v_ref, b_ref, o_ref):
        o_ref[...] = a_ref[...] + v_ref[0] + v_ref[1] + b_ref[...]

    return pl.pallas_call(
        body,
        grid=(NN // _BN,),
        in_specs=[
            pl.BlockSpec((_BN, D_OUT), lambda i: (i, 0)),
            pl.BlockSpec((NC, _BN, D_OUT), lambda i: (0, i, 0)),
            pl.BlockSpec((1, D_OUT), lambda i: (0, 0)),
        ],
        out_specs=pl.BlockSpec((_BN, D_OUT), lambda i: (i, 0)),
        out_shape=jax.ShapeDtypeStruct((NN, D_OUT), _F32),
    )(a2, v2, b2)


def kernel(x, edge_index, edge_weight, W1, b1, W2, b2):
    src = edge_index[0]
    dst = edge_index[1]

    w1a, w1b, w1c = W1[:D_IN], W1[D_IN:2 * D_IN], W1[2 * D_IN:]
    wcat1 = jnp.concatenate([w1a - w1c, w1b, w1c], axis=1)
    w2a, w2b, w2c = W2[:D_HID], W2[D_HID:2 * D_HID], W2[2 * D_HID:]
    wcat2 = jnp.concatenate([w2a - w2c, w2b, w2c], axis=1)

    a1, b1x, c1 = _tc_matmul_split(x, wcat1, D_IN, D_HID)
    norm, u = _sc1(src, dst, edge_weight, c1)
    v = _prop_second_64(src, dst, b1x, u, norm)
    a2, b2x, c2 = _tc_layer2_head(a1, v, b1.reshape(1, D_HID), wcat2)
    u2 = _prop_first_16(src, dst, c2, norm)
    v2 = _prop_second_16(src, dst, b2x, u2, norm)
    return _tc_final(a2, v2, b2.reshape(1, D_OUT))
